# baseline (speedup 1.0000x reference)
import functools

import jax
import jax.numpy as jnp
from jax import lax
from jax.experimental import pallas as pl
from jax.experimental.pallas import tpu as pltpu
from jax.experimental.pallas import tpu_sc as plsc

F32 = jnp.float32
NC = 2
NS = 16
NW = NC * NS
LN = 16
RB = 1000
KD = 80
KA = 112


def _sc_mesh():
    return plsc.VectorSubcoreMesh(core_axis_name="c", subcore_axis_name="s")


@functools.lru_cache(maxsize=None)
def _make_deg(N, NCH):
    RPS = N // NS

    def body(src_hbm, dst_hbm, out_hbm, srcb, dstb, obuf, zbuf, acc_o, acc_i,
             sem):
        c = lax.axis_index("c")
        s = lax.axis_index("s")
        wid = c * NS + s

        def fill_ones(i, carry):
            obuf[i, :] = jnp.ones((LN,), F32)
            return carry

        lax.fori_loop(0, KD, fill_ones, 0)

        def fill_zero(i, carry):
            zbuf[i, :] = jnp.zeros((LN,), F32)
            return carry

        lax.fori_loop(0, RPS, fill_zero, 0)
        pltpu.sync_copy(zbuf, acc_o.at[pl.ds(s * RPS, RPS)])
        pltpu.sync_copy(zbuf, acc_i.at[pl.ds(s * RPS, RPS)])
        plsc.subcore_barrier()

        pltpu.sync_copy(src_hbm.at[wid], srcb)
        pltpu.sync_copy(dst_hbm.at[wid], dstb)

        def step(t, carry):
            d = pltpu.async_copy(obuf, acc_o.at[srcb.at[t]], sem, add=True)
            pltpu.sync_copy(obuf, acc_i.at[dstb.at[t]], add=True)
            d.wait()
            return carry

        lax.fori_loop(0, NCH, step, 0)
        plsc.subcore_barrier()
        pltpu.sync_copy(acc_o.at[pl.ds(s * RPS, RPS)],
                        out_hbm.at[c, 0, pl.ds(s * RPS, RPS)])
        pltpu.sync_copy(acc_i.at[pl.ds(s * RPS, RPS)],
                        out_hbm.at[c, 1, pl.ds(s * RPS, RPS)])

    return pl.kernel(
        body,
        out_type=jax.ShapeDtypeStruct((NC, 2, N, LN), F32),
        mesh=_sc_mesh(),
        scratch_types=[
            pltpu.VMEM((NCH, KD), jnp.int32),
            pltpu.VMEM((NCH, KD), jnp.int32),
            pltpu.VMEM((KD, LN), F32),
            pltpu.VMEM((N // NS, LN), F32),
            pltpu.VMEM_SHARED((N, LN), F32),
            pltpu.VMEM_SHARED((N, LN), F32),
            pltpu.SemaphoreType.DMA,
        ],
        compiler_params=pltpu.CompilerParams(use_tc_tiling_on_sc=False),
    )


@functools.lru_cache(maxsize=None)
def _make_agg(N, H, NCH):
    NACC = N + LN
    assert NACC % NS == 0
    RPZ = NACC // NS
    ZFULL, ZREM = RPZ // KA, RPZ % KA
    RPS = N // NS

    def body(g_hbm, src_hbm, dst_hbm, out_hbm, srcb, dstb, rows, rows2,
             acc, sem, sem2):
        c = lax.axis_index("c")
        s = lax.axis_index("s")
        wid = c * NS + s

        def fill_zero(i, carry):
            for j in range(H // LN):
                rows[i, pl.ds(j * LN, LN)] = jnp.zeros((LN,), F32)
            return carry

        lax.fori_loop(0, KA, fill_zero, 0)
        for kk in range(ZFULL):
            pltpu.sync_copy(rows, acc.at[pl.ds(s * RPZ + kk * KA, KA)])
        if ZREM:
            pltpu.sync_copy(rows.at[pl.ds(0, ZREM)],
                            acc.at[pl.ds(s * RPZ + ZFULL * KA, ZREM)])
        plsc.subcore_barrier()

        pltpu.sync_copy(src_hbm.at[wid], srcb)
        pltpu.sync_copy(dst_hbm.at[wid], dstb)

        def gstart(t, buf, gsem):
            pltpu.async_copy(g_hbm.at[srcb.at[t]], buf, gsem)

        def gwait(buf, gsem):
            pltpu.make_async_copy(g_hbm.at[pl.ds(0, KA)], buf, gsem).wait()

        gstart(0, rows, sem)

        def pair(i, carry):
            tA = 2 * i
            gstart(tA + 1, rows2, sem2)
            gwait(rows, sem)
            pltpu.sync_copy(rows, acc.at[dstb.at[tA]], add=True)
            gstart(tA + 2, rows, sem)
            gwait(rows2, sem2)
            pltpu.sync_copy(rows2, acc.at[dstb.at[tA + 1]], add=True)
            return carry

        assert NCH % 2 == 1
        lax.fori_loop(0, (NCH - 1) // 2, pair, 0)
        gwait(rows, sem)
        pltpu.sync_copy(rows, acc.at[dstb.at[NCH - 1]], add=True)
        plsc.subcore_barrier()
        pltpu.sync_copy(acc.at[pl.ds(s * RPS, RPS)],
                        out_hbm.at[c, pl.ds(s * RPS, RPS)])

    return pl.kernel(
        body,
        out_type=jax.ShapeDtypeStruct((NC, N, H), F32),
        mesh=_sc_mesh(),
        scratch_types=[
            pltpu.VMEM((NCH, KA), jnp.int32),
            pltpu.VMEM((NCH, KA), jnp.int32),
            pltpu.VMEM((KA, H), F32),
            pltpu.VMEM((KA, H), F32),
            pltpu.VMEM_SHARED((NACC, H), F32),
            pltpu.SemaphoreType.DMA,
            pltpu.SemaphoreType.DMA,
        ],
        compiler_params=pltpu.CompilerParams(use_tc_tiling_on_sc=False),
    )


def _elu(a):
    return jnp.where(a > 0, a, jnp.exp(jnp.minimum(a, 0.0)) - 1.0)


def _tc_matmul(x, W):
    N, D = x.shape
    H = W.shape[1]

    def body(x_ref, w_ref, o_ref):
        o_ref[...] = jnp.dot(x_ref[...], w_ref[...],
                             preferred_element_type=F32)

    return pl.pallas_call(
        body,
        grid=(N // RB,),
        in_specs=[
            pl.BlockSpec((RB, D), lambda i: (i, 0)),
            pl.BlockSpec((D, H), lambda i: (0, 0)),
        ],
        out_specs=pl.BlockSpec((RB, H), lambda i: (i, 0)),
        out_shape=jax.ShapeDtypeStruct((N, H), F32),
    )(x, W)


def _tc_first(dp4, m0):
    _, N, _ = dp4.shape
    H = m0.shape[1]

    def body(dp_ref, m_ref, g_ref, cs_ref, cd_ref):
        p = dp_ref[...]
        deg_o = (p[0] + p[2])[:, :1]
        deg_i = (p[1] + p[3])[:, :1]
        cs = lax.rsqrt(jnp.maximum(deg_o, 1.0))
        cd = lax.rsqrt(jnp.maximum(deg_i, 1.0))
        cs_b = jnp.broadcast_to(cs, (RB, H))
        cs_ref[...] = cs_b
        cd_ref[...] = jnp.broadcast_to(cd, (RB, H))
        g_ref[...] = m_ref[...] * cs_b

    return pl.pallas_call(
        body,
        grid=(N // RB,),
        in_specs=[
            pl.BlockSpec((4, RB, LN), lambda i: (0, i, 0)),
            pl.BlockSpec((RB, H), lambda i: (i, 0)),
        ],
        out_specs=[
            pl.BlockSpec((RB, H), lambda i: (i, 0)),
            pl.BlockSpec((RB, H), lambda i: (i, 0)),
            pl.BlockSpec((RB, H), lambda i: (i, 0)),
        ],
        out_shape=[
            jax.ShapeDtypeStruct((N, H), F32),
            jax.ShapeDtypeStruct((N, H), F32),
            jax.ShapeDtypeStruct((N, H), F32),
        ],
    )(dp4, m0)


def _tc_layer(p, cd_b, b, W, cs_b):
    _, N, H = p.shape
    HO = W.shape[1]

    def body(p_ref, cd_ref, b_ref, w_ref, cs_ref, g_ref):
        a = (p_ref[0] + p_ref[1]) * cd_ref[...] + b_ref[...]
        a = _elu(a)
        g_ref[...] = jnp.dot(a, w_ref[...],
                             preferred_element_type=F32) * cs_ref[...]

    return pl.pallas_call(
        body,
        grid=(N // RB,),
        in_specs=[
            pl.BlockSpec((NC, RB, H), lambda i: (0, i, 0)),
            pl.BlockSpec((RB, H), lambda i: (i, 0)),
            pl.BlockSpec((1, H), lambda i: (0, 0)),
            pl.BlockSpec((H, HO), lambda i: (0, 0)),
            pl.BlockSpec((RB, HO), lambda i: (i, 0)),
        ],
        out_specs=pl.BlockSpec((RB, HO), lambda i: (i, 0)),
        out_shape=jax.ShapeDtypeStruct((N, HO), F32),
    )(p, cd_b, b.reshape(1, H), W, cs_b)


def _tc_final(p, cd_b, b, Wc, bc):
    _, N, H = p.shape
    C = Wc.shape[1]
    grid = N // RB

    def body(p_ref, cd_ref, b_ref, wc_ref, bc_ref, out_ref, acc_ref):
        i = pl.program_id(0)
        a = (p_ref[0] + p_ref[1]) * cd_ref[...] + b_ref[...]
        a = _elu(a)
        blk = jnp.sum(a, axis=0, keepdims=True)

        @pl.when(i == 0)
        def _init():
            acc_ref[...] = jnp.zeros_like(acc_ref)

        acc_ref[...] += blk

        @pl.when(i == grid - 1)
        def _fin():
            m = acc_ref[...] * (1.0 / N)
            out_ref[...] = jnp.dot(m, wc_ref[...],
                                   preferred_element_type=F32) + bc_ref[...]

    return pl.pallas_call(
        body,
        grid=(grid,),
        in_specs=[
            pl.BlockSpec((NC, RB, H), lambda i: (0, i, 0)),
            pl.BlockSpec((RB, H), lambda i: (i, 0)),
            pl.BlockSpec((1, H), lambda i: (0, 0)),
            pl.BlockSpec((H, C), lambda i: (0, 0)),
            pl.BlockSpec((1, C), lambda i: (0, 0)),
        ],
        out_specs=pl.BlockSpec((1, C), lambda i: (0, 0)),
        out_shape=jax.ShapeDtypeStruct((1, C), F32),
        scratch_shapes=[pltpu.VMEM((1, H), F32)],
    )(p, cd_b, b.reshape(1, H), Wc, bc.reshape(1, C))


def kernel(x, edge_index, W0, b0, W1, b1, W2, b2, Wc, bc):
    N, D = x.shape
    H = W0.shape[1]
    E = edge_index.shape[1]
    assert E % NW == 0 and N % NS == 0 and N % RB == 0

    PWD = E // NW
    assert PWD % KD == 0
    src_d = edge_index[0].reshape(NW, PWD // KD, KD)
    dst_d = edge_index[1].reshape(NW, PWD // KD, KD)

    NCH = (PWD + KA - 1) // KA
    if NCH % 2 == 0:
        NCH += 1
    PWP = NCH * KA
    pad = PWP - PWD
    dump = (N + jnp.arange(NW, dtype=jnp.int32) % NS)[:, None]
    src_a = jnp.pad(edge_index[0].reshape(NW, PWD),
                    ((0, 0), (0, pad))).reshape(NW, NCH, KA)
    dst_a = jnp.concatenate(
        [edge_index[1].reshape(NW, PWD),
         jnp.broadcast_to(dump, (NW, pad))], axis=1).reshape(NW, NCH, KA)

    dp = _make_deg(N, PWD // KD)(src_d, dst_d)
    m0 = _tc_matmul(x, W0)
    g, cs_b, cd_b = _tc_first(dp.reshape(NC * 2, N, LN), m0)

    agg = _make_agg(N, H, NCH)
    p = agg(g, src_a, dst_a)
    g = _tc_layer(p, cd_b, b0, W1, cs_b)
    p = agg(g, src_a, dst_a)
    g = _tc_layer(p, cd_b, b1, W2, cs_b)
    p = agg(g, src_a, dst_a)
    return _tc_final(p, cd_b, b2, Wc, bc)


# speedup vs baseline: 2.3887x; 2.3887x over previous
import functools

import jax
import jax.numpy as jnp
from jax import lax
from jax.experimental import pallas as pl
from jax.experimental.pallas import tpu as pltpu
from jax.experimental.pallas import tpu_sc as plsc

F32 = jnp.float32
NC = 2
NS = 16
NW = NC * NS
LN = 16
RB = 1000


def _sc_mesh():
    return plsc.VectorSubcoreMesh(core_axis_name="c", subcore_axis_name="s")


@functools.lru_cache(maxsize=None)
def _make_deg(N, NCH, K):
    RPS = N // NS

    def body(src_hbm, dst_hbm, out_hbm, srcb, dstb, obuf, zbuf, acc_o, acc_i,
             sem):
        c = lax.axis_index("c")
        s = lax.axis_index("s")
        wid = c * NS + s

        def fill_ones(i, carry):
            obuf[i, :] = jnp.ones((LN,), F32)
            return carry

        lax.fori_loop(0, K, fill_ones, 0)

        def fill_zero(i, carry):
            zbuf[i, :] = jnp.zeros((LN,), F32)
            return carry

        lax.fori_loop(0, RPS, fill_zero, 0)
        pltpu.sync_copy(zbuf, acc_o.at[pl.ds(s * RPS, RPS)])
        pltpu.sync_copy(zbuf, acc_i.at[pl.ds(s * RPS, RPS)])
        plsc.subcore_barrier()

        pltpu.sync_copy(src_hbm.at[wid], srcb)
        pltpu.sync_copy(dst_hbm.at[wid], dstb)

        def step(t, carry):
            d = pltpu.async_copy(obuf, acc_o.at[srcb.at[t]], sem, add=True)
            pltpu.sync_copy(obuf, acc_i.at[dstb.at[t]], add=True)
            d.wait()
            return carry

        lax.fori_loop(0, NCH, step, 0)
        plsc.subcore_barrier()
        pltpu.sync_copy(acc_o.at[pl.ds(s * RPS, RPS)],
                        out_hbm.at[c, 0, pl.ds(s * RPS, RPS)])
        pltpu.sync_copy(acc_i.at[pl.ds(s * RPS, RPS)],
                        out_hbm.at[c, 1, pl.ds(s * RPS, RPS)])

    return pl.kernel(
        body,
        out_type=jax.ShapeDtypeStruct((NC, 2, N, LN), F32),
        mesh=_sc_mesh(),
        scratch_types=[
            pltpu.VMEM((NCH, K), jnp.int32),
            pltpu.VMEM((NCH, K), jnp.int32),
            pltpu.VMEM((K, LN), F32),
            pltpu.VMEM((RPS, LN), F32),
            pltpu.VMEM_SHARED((N, LN), F32),
            pltpu.VMEM_SHARED((N, LN), F32),
            pltpu.SemaphoreType.DMA,
        ],
        compiler_params=pltpu.CompilerParams(use_tc_tiling_on_sc=False),
    )


@functools.lru_cache(maxsize=None)
def _make_agg(N, H, NCH, K):
    RPS = N // NS
    assert NCH % 2 == 1
    ZFULL, ZREM = RPS // K, RPS % K

    def body(g_hbm, src_hbm, dst_hbm, out_hbm, srcb, dstb, rows, rows2,
             acc, sem, sem2):
        c = lax.axis_index("c")
        s = lax.axis_index("s")
        wid = c * NS + s

        def fill_zero(i, carry):
            for j in range(H // LN):
                rows[i, pl.ds(j * LN, LN)] = jnp.zeros((LN,), F32)
            return carry

        lax.fori_loop(0, K, fill_zero, 0)
        for kk in range(ZFULL):
            pltpu.sync_copy(rows, acc.at[pl.ds(s * RPS + kk * K, K)])
        if ZREM:
            pltpu.sync_copy(rows.at[pl.ds(0, ZREM)],
                            acc.at[pl.ds(s * RPS + ZFULL * K, ZREM)])
        plsc.subcore_barrier()

        pltpu.sync_copy(src_hbm.at[wid], srcb)
        pltpu.sync_copy(dst_hbm.at[wid], dstb)

        def gstart(t, buf, gsem):
            pltpu.async_copy(g_hbm.at[srcb.at[t]], buf, gsem)

        def gwait(buf, gsem):
            pltpu.make_async_copy(g_hbm.at[pl.ds(0, K)], buf, gsem).wait()

        gstart(0, rows, sem)

        def pair(i, carry):
            tA = 2 * i
            gstart(tA + 1, rows2, sem2)
            gwait(rows, sem)
            pltpu.sync_copy(rows, acc.at[dstb.at[tA]], add=True)
            gstart(tA + 2, rows, sem)
            gwait(rows2, sem2)
            pltpu.sync_copy(rows2, acc.at[dstb.at[tA + 1]], add=True)
            return carry

        lax.fori_loop(0, (NCH - 1) // 2, pair, 0)
        gwait(rows, sem)
        pltpu.sync_copy(rows, acc.at[dstb.at[NCH - 1]], add=True)
        plsc.subcore_barrier()
        pltpu.sync_copy(acc.at[pl.ds(s * RPS, RPS)],
                        out_hbm.at[c, pl.ds(s * RPS, RPS)])

    return pl.kernel(
        body,
        out_type=jax.ShapeDtypeStruct((NC, N, H), F32),
        mesh=_sc_mesh(),
        scratch_types=[
            pltpu.VMEM((NCH, K), jnp.int32),
            pltpu.VMEM((NCH, K), jnp.int32),
            pltpu.VMEM((K, H), F32),
            pltpu.VMEM((K, H), F32),
            pltpu.VMEM_SHARED((N, H), F32),
            pltpu.SemaphoreType.DMA,
            pltpu.SemaphoreType.DMA,
        ],
        compiler_params=pltpu.CompilerParams(use_tc_tiling_on_sc=False),
    )


def _elu(a):
    return jnp.where(a > 0, a, jnp.exp(jnp.minimum(a, 0.0)) - 1.0)


def _tc_matmul(x, W):
    N, D = x.shape
    H = W.shape[1]

    def body(x_ref, w_ref, o_ref):
        o_ref[...] = jnp.dot(x_ref[...], w_ref[...],
                             preferred_element_type=F32)

    return pl.pallas_call(
        body,
        grid=(N // RB,),
        in_specs=[
            pl.BlockSpec((RB, D), lambda i: (i, 0)),
            pl.BlockSpec((D, H), lambda i: (0, 0)),
        ],
        out_specs=pl.BlockSpec((RB, H), lambda i: (i, 0)),
        out_shape=jax.ShapeDtypeStruct((N, H), F32),
    )(x, W)


def _tc_first(dp4, m0):
    _, N, _ = dp4.shape
    H = m0.shape[1]
    grid = N // RB

    def body(dp_ref, m_ref, g_ref, cs_ref, cd_ref):
        p = dp_ref[...]
        deg_o = (p[0] + p[2])[:, :1]
        deg_i = (p[1] + p[3])[:, :1]
        cs = lax.rsqrt(jnp.maximum(deg_o, 1.0))
        cd = lax.rsqrt(jnp.maximum(deg_i, 1.0))
        cs_b = jnp.broadcast_to(cs, (RB, H))
        cd_b = jnp.broadcast_to(cd, (RB, H))
        cs_ref[...] = cs_b
        cd_ref[...] = cd_b
        g_ref[...] = m_ref[...] * cs_b

    return pl.pallas_call(
        body,
        grid=(grid,),
        in_specs=[
            pl.BlockSpec((4, RB, LN), lambda i: (0, i, 0)),
            pl.BlockSpec((RB, H), lambda i: (i, 0)),
        ],
        out_specs=[
            pl.BlockSpec((RB, H), lambda i: (i, 0)),
            pl.BlockSpec((RB, H), lambda i: (i, 0)),
            pl.BlockSpec((RB, H), lambda i: (i, 0)),
        ],
        out_shape=[
            jax.ShapeDtypeStruct((N, H), F32),
            jax.ShapeDtypeStruct((N, H), F32),
            jax.ShapeDtypeStruct((N, H), F32),
        ],
    )(dp4, m0)


def _tc_layer(p, cd_b, b, W, cs_b):
    _, N, H = p.shape
    HO = W.shape[1]
    grid = N // RB

    def body(p_ref, cd_ref, b_ref, w_ref, cs_ref, g_ref):
        a = (p_ref[0] + p_ref[1]) * cd_ref[...] + b_ref[...]
        a = _elu(a)
        g_ref[...] = jnp.dot(a, w_ref[...],
                             preferred_element_type=F32) * cs_ref[...]

    return pl.pallas_call(
        body,
        grid=(grid,),
        in_specs=[
            pl.BlockSpec((NC, RB, H), lambda i: (0, i, 0)),
            pl.BlockSpec((RB, H), lambda i: (i, 0)),
            pl.BlockSpec((1, H), lambda i: (0, 0)),
            pl.BlockSpec((H, HO), lambda i: (0, 0)),
            pl.BlockSpec((RB, HO), lambda i: (i, 0)),
        ],
        out_specs=pl.BlockSpec((RB, HO), lambda i: (i, 0)),
        out_shape=jax.ShapeDtypeStruct((N, HO), F32),
    )(p, cd_b, b.reshape(1, H), W, cs_b)


def _tc_final(p, cd_b, b, Wc, bc):
    _, N, H = p.shape
    C = Wc.shape[1]
    grid = N // RB

    def body(p_ref, cd_ref, b_ref, wc_ref, bc_ref, out_ref, acc_ref):
        i = pl.program_id(0)
        a = (p_ref[0] + p_ref[1]) * cd_ref[...] + b_ref[...]
        a = _elu(a)
        blk = jnp.sum(a, axis=0, keepdims=True)

        @pl.when(i == 0)
        def _init():
            acc_ref[...] = jnp.zeros_like(acc_ref)

        acc_ref[...] += blk

        @pl.when(i == grid - 1)
        def _fin():
            m = acc_ref[...] * (1.0 / N)
            out_ref[...] = jnp.dot(m, wc_ref[...],
                                   preferred_element_type=F32) + bc_ref[...]

    return pl.pallas_call(
        body,
        grid=(grid,),
        in_specs=[
            pl.BlockSpec((NC, RB, H), lambda i: (0, i, 0)),
            pl.BlockSpec((RB, H), lambda i: (i, 0)),
            pl.BlockSpec((1, H), lambda i: (0, 0)),
            pl.BlockSpec((H, C), lambda i: (0, 0)),
            pl.BlockSpec((1, C), lambda i: (0, 0)),
        ],
        out_specs=pl.BlockSpec((1, C), lambda i: (0, 0)),
        out_shape=jax.ShapeDtypeStruct((1, C), F32),
        scratch_shapes=[pltpu.VMEM((1, H), F32)],
    )(p, cd_b, b.reshape(1, H), Wc, bc.reshape(1, C))


def kernel(x, edge_index, W0, b0, W1, b1, W2, b2, Wc, bc):
    N, D = x.shape
    H = W0.shape[1]
    E = edge_index.shape[1]
    assert E % NW == 0 and N % NS == 0 and N % RB == 0
    PW = E // NW
    K = 80
    assert PW % K == 0
    NCH = PW // K

    src = edge_index[0].reshape(NW, NCH, K)
    dst = edge_index[1].reshape(NW, NCH, K)

    dp = _make_deg(N, NCH, K)(src, dst)
    m0 = _tc_matmul(x, W0)
    g, cs_b, cd_b = _tc_first(dp.reshape(NC * 2, N, LN), m0)

    agg = _make_agg(N, H, NCH, K)
    p = agg(g, src, dst)
    g = _tc_layer(p, cd_b, b0, W1, cs_b)
    p = agg(g, src, dst)
    g = _tc_layer(p, cd_b, b1, W2, cs_b)
    p = agg(g, src, dst)
    return _tc_final(p, cd_b, b2, Wc, bc)


# speedup vs baseline: 2.8186x; 1.1800x over previous
import functools

import jax
import jax.numpy as jnp
from jax import lax
from jax.experimental import pallas as pl
from jax.experimental.pallas import tpu as pltpu
from jax.experimental.pallas import tpu_sc as plsc

F32 = jnp.float32
NC = 2
NS = 16
NW = NC * NS
LN = 16
RB = 1000


def _sc_mesh():
    return plsc.VectorSubcoreMesh(core_axis_name="c", subcore_axis_name="s")


@functools.lru_cache(maxsize=None)
def _make_deg(N, NCH, K):
    RPS = N // NS

    def body(src_hbm, dst_hbm, out_hbm, srcb, dstb, obuf, zbuf, acc_o, acc_i,
             sem, sem2):
        c = lax.axis_index("c")
        s = lax.axis_index("s")
        wid = c * NS + s

        def fill_ones(i, carry):
            obuf[i, :] = jnp.ones((LN,), F32)
            return carry

        lax.fori_loop(0, K, fill_ones, 0)

        def fill_zero(i, carry):
            zbuf[i, :] = jnp.zeros((LN,), F32)
            return carry

        lax.fori_loop(0, RPS, fill_zero, 0)
        pltpu.sync_copy(zbuf, acc_o.at[pl.ds(s * RPS, RPS)])
        pltpu.sync_copy(zbuf, acc_i.at[pl.ds(s * RPS, RPS)])
        plsc.subcore_barrier()

        pltpu.sync_copy(src_hbm.at[wid], srcb)
        pltpu.sync_copy(dst_hbm.at[wid], dstb)

        DB = 5
        assert NCH % DB == 0

        def step(i, carry):
            ds = []
            for j in range(DB):
                t = i * DB + j
                ds.append(pltpu.async_copy(obuf, acc_o.at[srcb.at[t]], sem,
                                           add=True))
                ds.append(pltpu.async_copy(obuf, acc_i.at[dstb.at[t]], sem2,
                                           add=True))
            for d in ds:
                d.wait()
            return carry

        lax.fori_loop(0, NCH // DB, step, 0)
        plsc.subcore_barrier()
        pltpu.sync_copy(acc_o.at[pl.ds(s * RPS, RPS)],
                        out_hbm.at[c, 0, pl.ds(s * RPS, RPS)])
        pltpu.sync_copy(acc_i.at[pl.ds(s * RPS, RPS)],
                        out_hbm.at[c, 1, pl.ds(s * RPS, RPS)])

    return pl.kernel(
        body,
        out_type=jax.ShapeDtypeStruct((NC, 2, N, LN), F32),
        mesh=_sc_mesh(),
        scratch_types=[
            pltpu.VMEM((NCH, K), jnp.int32),
            pltpu.VMEM((NCH, K), jnp.int32),
            pltpu.VMEM((K, LN), F32),
            pltpu.VMEM((RPS, LN), F32),
            pltpu.VMEM_SHARED((N, LN), F32),
            pltpu.VMEM_SHARED((N, LN), F32),
            pltpu.SemaphoreType.DMA,
            pltpu.SemaphoreType.DMA,
        ],
        compiler_params=pltpu.CompilerParams(use_tc_tiling_on_sc=False),
    )


@functools.lru_cache(maxsize=None)
def _make_agg(N, H, NCH, K):
    RPS = N // NS
    assert NCH % 2 == 1
    ZFULL, ZREM = RPS // K, RPS % K

    def body(g_hbm, src_hbm, dst_hbm, out_hbm, srcb, dstb, rows, rows2, rows3,
             acc, sem, sem2, sem3):
        c = lax.axis_index("c")
        s = lax.axis_index("s")
        wid = c * NS + s

        def fill_zero(i, carry):
            for j in range(H // LN):
                rows[i, pl.ds(j * LN, LN)] = jnp.zeros((LN,), F32)
            return carry

        lax.fori_loop(0, K, fill_zero, 0)
        for kk in range(ZFULL):
            pltpu.sync_copy(rows, acc.at[pl.ds(s * RPS + kk * K, K)])
        if ZREM:
            pltpu.sync_copy(rows.at[pl.ds(0, ZREM)],
                            acc.at[pl.ds(s * RPS + ZFULL * K, ZREM)])
        plsc.subcore_barrier()

        pltpu.sync_copy(src_hbm.at[wid], srcb)
        pltpu.sync_copy(dst_hbm.at[wid], dstb)

        def gstart(t, buf, gsem):
            pltpu.async_copy(g_hbm.at[srcb.at[t]], buf, gsem)

        def gwait(buf, gsem):
            pltpu.make_async_copy(g_hbm.at[pl.ds(0, K)], buf, gsem).wait()

        assert (NCH - 2) % 3 == 0
        gstart(0, rows, sem)
        gstart(1, rows2, sem2)

        def triple(i, carry):
            t = 3 * i
            gstart(t + 2, rows3, sem3)
            gwait(rows, sem)
            pltpu.sync_copy(rows, acc.at[dstb.at[t]], add=True)
            gstart(t + 3, rows, sem)
            gwait(rows2, sem2)
            pltpu.sync_copy(rows2, acc.at[dstb.at[t + 1]], add=True)
            gstart(t + 4, rows2, sem2)
            gwait(rows3, sem3)
            pltpu.sync_copy(rows3, acc.at[dstb.at[t + 2]], add=True)
            return carry

        lax.fori_loop(0, (NCH - 2) // 3, triple, 0)
        gwait(rows, sem)
        pltpu.sync_copy(rows, acc.at[dstb.at[NCH - 2]], add=True)
        gwait(rows2, sem2)
        pltpu.sync_copy(rows2, acc.at[dstb.at[NCH - 1]], add=True)
        plsc.subcore_barrier()
        pltpu.sync_copy(acc.at[pl.ds(s * RPS, RPS)],
                        out_hbm.at[c, pl.ds(s * RPS, RPS)])

    return pl.kernel(
        body,
        out_type=jax.ShapeDtypeStruct((NC, N, H), F32),
        mesh=_sc_mesh(),
        scratch_types=[
            pltpu.VMEM((NCH, K), jnp.int32),
            pltpu.VMEM((NCH, K), jnp.int32),
            pltpu.VMEM((K, H), F32),
            pltpu.VMEM((K, H), F32),
            pltpu.VMEM((K, H), F32),
            pltpu.VMEM_SHARED((N, H), F32),
            pltpu.SemaphoreType.DMA,
            pltpu.SemaphoreType.DMA,
            pltpu.SemaphoreType.DMA,
        ],
        compiler_params=pltpu.CompilerParams(use_tc_tiling_on_sc=False),
    )


def _elu(a):
    return jnp.where(a > 0, a, jnp.exp(jnp.minimum(a, 0.0)) - 1.0)


def _tc_matmul(x, W):
    N, D = x.shape
    H = W.shape[1]

    def body(x_ref, w_ref, o_ref):
        o_ref[...] = jnp.dot(x_ref[...], w_ref[...],
                             preferred_element_type=F32)

    return pl.pallas_call(
        body,
        grid=(N // RB,),
        in_specs=[
            pl.BlockSpec((RB, D), lambda i: (i, 0)),
            pl.BlockSpec((D, H), lambda i: (0, 0)),
        ],
        out_specs=pl.BlockSpec((RB, H), lambda i: (i, 0)),
        out_shape=jax.ShapeDtypeStruct((N, H), F32),
    )(x, W)


def _tc_first(dp4, m0):
    _, N, _ = dp4.shape
    H = m0.shape[1]
    grid = N // RB

    def body(dp_ref, m_ref, g_ref, cs_ref, cd_ref):
        p = dp_ref[...]
        deg_o = (p[0] + p[2])[:, :1]
        deg_i = (p[1] + p[3])[:, :1]
        cs = lax.rsqrt(jnp.maximum(deg_o, 1.0))
        cd = lax.rsqrt(jnp.maximum(deg_i, 1.0))
        cs_b = jnp.broadcast_to(cs, (RB, H))
        cd_b = jnp.broadcast_to(cd, (RB, H))
        cs_ref[...] = cs_b
        cd_ref[...] = cd_b
        g_ref[...] = m_ref[...] * cs_b

    return pl.pallas_call(
        body,
        grid=(grid,),
        in_specs=[
            pl.BlockSpec((4, RB, LN), lambda i: (0, i, 0)),
            pl.BlockSpec((RB, H), lambda i: (i, 0)),
        ],
        out_specs=[
            pl.BlockSpec((RB, H), lambda i: (i, 0)),
            pl.BlockSpec((RB, H), lambda i: (i, 0)),
            pl.BlockSpec((RB, H), lambda i: (i, 0)),
        ],
        out_shape=[
            jax.ShapeDtypeStruct((N, H), F32),
            jax.ShapeDtypeStruct((N, H), F32),
            jax.ShapeDtypeStruct((N, H), F32),
        ],
    )(dp4, m0)


def _tc_layer(p, cd_b, b, W, cs_b):
    _, N, H = p.shape
    HO = W.shape[1]
    grid = N // RB

    def body(p_ref, cd_ref, b_ref, w_ref, cs_ref, g_ref):
        a = (p_ref[0] + p_ref[1]) * cd_ref[...] + b_ref[...]
        a = _elu(a)
        g_ref[...] = jnp.dot(a, w_ref[...],
                             preferred_element_type=F32) * cs_ref[...]

    return pl.pallas_call(
        body,
        grid=(grid,),
        in_specs=[
            pl.BlockSpec((NC, RB, H), lambda i: (0, i, 0)),
            pl.BlockSpec((RB, H), lambda i: (i, 0)),
            pl.BlockSpec((1, H), lambda i: (0, 0)),
            pl.BlockSpec((H, HO), lambda i: (0, 0)),
            pl.BlockSpec((RB, HO), lambda i: (i, 0)),
        ],
        out_specs=pl.BlockSpec((RB, HO), lambda i: (i, 0)),
        out_shape=jax.ShapeDtypeStruct((N, HO), F32),
    )(p, cd_b, b.reshape(1, H), W, cs_b)


def _tc_final(p, cd_b, b, Wc, bc):
    _, N, H = p.shape
    C = Wc.shape[1]
    grid = N // RB

    def body(p_ref, cd_ref, b_ref, wc_ref, bc_ref, out_ref, acc_ref):
        i = pl.program_id(0)
        a = (p_ref[0] + p_ref[1]) * cd_ref[...] + b_ref[...]
        a = _elu(a)
        blk = jnp.sum(a, axis=0, keepdims=True)

        @pl.when(i == 0)
        def _init():
            acc_ref[...] = jnp.zeros_like(acc_ref)

        acc_ref[...] += blk

        @pl.when(i == grid - 1)
        def _fin():
            m = acc_ref[...] * (1.0 / N)
            out_ref[...] = jnp.dot(m, wc_ref[...],
                                   preferred_element_type=F32) + bc_ref[...]

    return pl.pallas_call(
        body,
        grid=(grid,),
        in_specs=[
            pl.BlockSpec((NC, RB, H), lambda i: (0, i, 0)),
            pl.BlockSpec((RB, H), lambda i: (i, 0)),
            pl.BlockSpec((1, H), lambda i: (0, 0)),
            pl.BlockSpec((H, C), lambda i: (0, 0)),
            pl.BlockSpec((1, C), lambda i: (0, 0)),
        ],
        out_specs=pl.BlockSpec((1, C), lambda i: (0, 0)),
        out_shape=jax.ShapeDtypeStruct((1, C), F32),
        scratch_shapes=[pltpu.VMEM((1, H), F32)],
    )(p, cd_b, b.reshape(1, H), Wc, bc.reshape(1, C))


def kernel(x, edge_index, W0, b0, W1, b1, W2, b2, Wc, bc):
    N, D = x.shape
    H = W0.shape[1]
    E = edge_index.shape[1]
    assert E % NW == 0 and N % NS == 0 and N % RB == 0
    PW = E // NW
    K = 80
    assert PW % K == 0
    NCH = PW // K

    src = edge_index[0].reshape(NW, NCH, K)
    dst = edge_index[1].reshape(NW, NCH, K)

    dp = _make_deg(N, NCH, K)(src, dst)
    m0 = _tc_matmul(x, W0)
    g, cs_b, cd_b = _tc_first(dp.reshape(NC * 2, N, LN), m0)

    agg = _make_agg(N, H, NCH, K)
    p = agg(g, src, dst)
    g = _tc_layer(p, cd_b, b0, W1, cs_b)
    p = agg(g, src, dst)
    g = _tc_layer(p, cd_b, b1, W2, cs_b)
    p = agg(g, src, dst)
    return _tc_final(p, cd_b, b2, Wc, bc)


# speedup vs baseline: 2.8577x; 1.0139x over previous
import functools

import jax
import jax.numpy as jnp
from jax import lax
from jax.experimental import pallas as pl
from jax.experimental.pallas import tpu as pltpu
from jax.experimental.pallas import tpu_sc as plsc

F32 = jnp.float32
BF16 = jnp.bfloat16
NC = 2
NS = 16
NW = NC * NS
LN = 16
RB = 1000


def _sc_mesh():
    return plsc.VectorSubcoreMesh(core_axis_name="c", subcore_axis_name="s")


@functools.lru_cache(maxsize=None)
def _make_deg(N, NCH, K):
    RPS = N // NS

    def body(src_hbm, dst_hbm, out_hbm, srcb, dstb, obuf, zbuf, acc_o, acc_i,
             sem, sem2):
        c = lax.axis_index("c")
        s = lax.axis_index("s")
        wid = c * NS + s

        def fill_ones(i, carry):
            obuf[i, :] = jnp.ones((LN,), F32)
            return carry

        lax.fori_loop(0, K, fill_ones, 0)

        def fill_zero(i, carry):
            zbuf[i, :] = jnp.zeros((LN,), F32)
            return carry

        lax.fori_loop(0, RPS, fill_zero, 0)
        pltpu.sync_copy(zbuf, acc_o.at[pl.ds(s * RPS, RPS)])
        pltpu.sync_copy(zbuf, acc_i.at[pl.ds(s * RPS, RPS)])
        plsc.subcore_barrier()

        pltpu.sync_copy(src_hbm.at[wid], srcb)
        pltpu.sync_copy(dst_hbm.at[wid], dstb)

        DB = 5
        assert NCH % DB == 0

        def step(i, carry):
            ds = []
            for j in range(DB):
                t = i * DB + j
                ds.append(pltpu.async_copy(obuf, acc_o.at[srcb.at[t]], sem,
                                           add=True))
                ds.append(pltpu.async_copy(obuf, acc_i.at[dstb.at[t]], sem2,
                                           add=True))
            for d in ds:
                d.wait()
            return carry

        lax.fori_loop(0, NCH // DB, step, 0)
        plsc.subcore_barrier()
        pltpu.sync_copy(acc_o.at[pl.ds(s * RPS, RPS)],
                        out_hbm.at[c, 0, pl.ds(s * RPS, RPS)])
        pltpu.sync_copy(acc_i.at[pl.ds(s * RPS, RPS)],
                        out_hbm.at[c, 1, pl.ds(s * RPS, RPS)])

    return pl.kernel(
        body,
        out_type=jax.ShapeDtypeStruct((NC, 2, N, LN), F32),
        mesh=_sc_mesh(),
        scratch_types=[
            pltpu.VMEM((NCH, K), jnp.int32),
            pltpu.VMEM((NCH, K), jnp.int32),
            pltpu.VMEM((K, LN), F32),
            pltpu.VMEM((RPS, LN), F32),
            pltpu.VMEM_SHARED((N, LN), F32),
            pltpu.VMEM_SHARED((N, LN), F32),
            pltpu.SemaphoreType.DMA,
            pltpu.SemaphoreType.DMA,
        ],
        compiler_params=pltpu.CompilerParams(use_tc_tiling_on_sc=False),
    )


@functools.lru_cache(maxsize=None)
def _make_agg(N, H, NCH, K):
    RPS = N // NS
    assert NCH % 2 == 1
    ZFULL, ZREM = RPS // K, RPS % K

    def body(g_hbm, src_hbm, dst_hbm, out_hbm, srcb, dstb, rows, rows2, rows3,
             acc, sem, sem2, sem3):
        c = lax.axis_index("c")
        s = lax.axis_index("s")
        wid = c * NS + s

        def fill_zero(i, carry):
            for j in range(H // (2 * LN)):
                rows[i, pl.ds(j * 2 * LN, 2 * LN)] = jnp.zeros((2 * LN,), BF16)
            return carry

        lax.fori_loop(0, K, fill_zero, 0)
        for kk in range(ZFULL):
            pltpu.sync_copy(rows, acc.at[pl.ds(s * RPS + kk * K, K)])
        if ZREM:
            pltpu.sync_copy(rows.at[pl.ds(0, ZREM)],
                            acc.at[pl.ds(s * RPS + ZFULL * K, ZREM)])
        plsc.subcore_barrier()

        pltpu.sync_copy(src_hbm.at[wid], srcb)
        pltpu.sync_copy(dst_hbm.at[wid], dstb)

        def gstart(t, buf, gsem):
            pltpu.async_copy(g_hbm.at[srcb.at[t]], buf, gsem)

        def gwait(buf, gsem):
            pltpu.make_async_copy(g_hbm.at[pl.ds(0, K)], buf, gsem).wait()

        assert (NCH - 2) % 3 == 0
        gstart(0, rows, sem)
        gstart(1, rows2, sem2)

        def triple(i, carry):
            t = 3 * i
            gstart(t + 2, rows3, sem3)
            gwait(rows, sem)
            pltpu.sync_copy(rows, acc.at[dstb.at[t]], add=True)
            gstart(t + 3, rows, sem)
            gwait(rows2, sem2)
            pltpu.sync_copy(rows2, acc.at[dstb.at[t + 1]], add=True)
            gstart(t + 4, rows2, sem2)
            gwait(rows3, sem3)
            pltpu.sync_copy(rows3, acc.at[dstb.at[t + 2]], add=True)
            return carry

        lax.fori_loop(0, (NCH - 2) // 3, triple, 0)
        gwait(rows, sem)
        pltpu.sync_copy(rows, acc.at[dstb.at[NCH - 2]], add=True)
        gwait(rows2, sem2)
        pltpu.sync_copy(rows2, acc.at[dstb.at[NCH - 1]], add=True)
        plsc.subcore_barrier()
        pltpu.sync_copy(acc.at[pl.ds(s * RPS, RPS)],
                        out_hbm.at[c, pl.ds(s * RPS, RPS)])

    return pl.kernel(
        body,
        out_type=jax.ShapeDtypeStruct((NC, N, H), BF16),
        mesh=_sc_mesh(),
        scratch_types=[
            pltpu.VMEM((NCH, K), jnp.int32),
            pltpu.VMEM((NCH, K), jnp.int32),
            pltpu.VMEM((K, H), BF16),
            pltpu.VMEM((K, H), BF16),
            pltpu.VMEM((K, H), BF16),
            pltpu.VMEM_SHARED((N, H), BF16),
            pltpu.SemaphoreType.DMA,
            pltpu.SemaphoreType.DMA,
            pltpu.SemaphoreType.DMA,
        ],
        compiler_params=pltpu.CompilerParams(use_tc_tiling_on_sc=False),
    )


def _elu(a):
    return jnp.where(a > 0, a, jnp.exp(jnp.minimum(a, 0.0)) - 1.0)


def _tc_matmul(x, W):
    N, D = x.shape
    H = W.shape[1]

    def body(x_ref, w_ref, o_ref):
        o_ref[...] = jnp.dot(x_ref[...], w_ref[...],
                             preferred_element_type=F32)

    return pl.pallas_call(
        body,
        grid=(N // RB,),
        in_specs=[
            pl.BlockSpec((RB, D), lambda i: (i, 0)),
            pl.BlockSpec((D, H), lambda i: (0, 0)),
        ],
        out_specs=pl.BlockSpec((RB, H), lambda i: (i, 0)),
        out_shape=jax.ShapeDtypeStruct((N, H), F32),
    )(x, W)


def _tc_first(dp4, m0):
    _, N, _ = dp4.shape
    H = m0.shape[1]
    grid = N // RB

    def body(dp_ref, m_ref, g_ref, cs_ref, cd_ref):
        p = dp_ref[...]
        deg_o = (p[0] + p[2])[:, :1]
        deg_i = (p[1] + p[3])[:, :1]
        cs = lax.rsqrt(jnp.maximum(deg_o, 1.0))
        cd = lax.rsqrt(jnp.maximum(deg_i, 1.0))
        cs_b = jnp.broadcast_to(cs, (RB, H))
        cd_b = jnp.broadcast_to(cd, (RB, H))
        cs_ref[...] = cs_b
        cd_ref[...] = cd_b
        g_ref[...] = (m_ref[...] * cs_b).astype(BF16)

    return pl.pallas_call(
        body,
        grid=(grid,),
        in_specs=[
            pl.BlockSpec((4, RB, LN), lambda i: (0, i, 0)),
            pl.BlockSpec((RB, H), lambda i: (i, 0)),
        ],
        out_specs=[
            pl.BlockSpec((RB, H), lambda i: (i, 0)),
            pl.BlockSpec((RB, H), lambda i: (i, 0)),
            pl.BlockSpec((RB, H), lambda i: (i, 0)),
        ],
        out_shape=[
            jax.ShapeDtypeStruct((N, H), BF16),
            jax.ShapeDtypeStruct((N, H), F32),
            jax.ShapeDtypeStruct((N, H), F32),
        ],
    )(dp4, m0)


def _tc_layer(p, cd_b, b, W, cs_b):
    _, N, H = p.shape
    HO = W.shape[1]
    grid = N // RB

    def body(p_ref, cd_ref, b_ref, w_ref, cs_ref, g_ref):
        psum = p_ref[0].astype(F32) + p_ref[1].astype(F32)
        a = psum * cd_ref[...] + b_ref[...]
        a = _elu(a)
        g_ref[...] = (jnp.dot(a, w_ref[...],
                              preferred_element_type=F32)
                      * cs_ref[...]).astype(BF16)

    return pl.pallas_call(
        body,
        grid=(grid,),
        in_specs=[
            pl.BlockSpec((NC, RB, H), lambda i: (0, i, 0)),
            pl.BlockSpec((RB, H), lambda i: (i, 0)),
            pl.BlockSpec((1, H), lambda i: (0, 0)),
            pl.BlockSpec((H, HO), lambda i: (0, 0)),
            pl.BlockSpec((RB, HO), lambda i: (i, 0)),
        ],
        out_specs=pl.BlockSpec((RB, HO), lambda i: (i, 0)),
        out_shape=jax.ShapeDtypeStruct((N, HO), BF16),
    )(p, cd_b, b.reshape(1, H), W, cs_b)


def _tc_final(p, cd_b, b, Wc, bc):
    _, N, H = p.shape
    C = Wc.shape[1]
    grid = N // RB

    def body(p_ref, cd_ref, b_ref, wc_ref, bc_ref, out_ref, acc_ref):
        i = pl.program_id(0)
        psum = p_ref[0].astype(F32) + p_ref[1].astype(F32)
        a = psum * cd_ref[...] + b_ref[...]
        a = _elu(a)
        blk = jnp.sum(a, axis=0, keepdims=True)

        @pl.when(i == 0)
        def _init():
            acc_ref[...] = jnp.zeros_like(acc_ref)

        acc_ref[...] += blk

        @pl.when(i == grid - 1)
        def _fin():
            m = acc_ref[...] * (1.0 / N)
            out_ref[...] = jnp.dot(m, wc_ref[...],
                                   preferred_element_type=F32) + bc_ref[...]

    return pl.pallas_call(
        body,
        grid=(grid,),
        in_specs=[
            pl.BlockSpec((NC, RB, H), lambda i: (0, i, 0)),
            pl.BlockSpec((RB, H), lambda i: (i, 0)),
            pl.BlockSpec((1, H), lambda i: (0, 0)),
            pl.BlockSpec((H, C), lambda i: (0, 0)),
            pl.BlockSpec((1, C), lambda i: (0, 0)),
        ],
        out_specs=pl.BlockSpec((1, C), lambda i: (0, 0)),
        out_shape=jax.ShapeDtypeStruct((1, C), F32),
        scratch_shapes=[pltpu.VMEM((1, H), F32)],
    )(p, cd_b, b.reshape(1, H), Wc, bc.reshape(1, C))


def kernel(x, edge_index, W0, b0, W1, b1, W2, b2, Wc, bc):
    N, D = x.shape
    H = W0.shape[1]
    E = edge_index.shape[1]
    assert E % NW == 0 and N % NS == 0 and N % RB == 0
    PW = E // NW
    K = 80
    assert PW % K == 0
    NCH = PW // K

    src = edge_index[0].reshape(NW, NCH, K)
    dst = edge_index[1].reshape(NW, NCH, K)

    dp = _make_deg(N, NCH, K)(src, dst)
    m0 = _tc_matmul(x, W0)
    g, cs_b, cd_b = _tc_first(dp.reshape(NC * 2, N, LN), m0)

    agg = _make_agg(N, H, NCH, K)
    p = agg(g, src, dst)
    g = _tc_layer(p, cd_b, b0, W1, cs_b)
    p = agg(g, src, dst)
    g = _tc_layer(p, cd_b, b1, W2, cs_b)
    p = agg(g, src, dst)
    return _tc_final(p, cd_b, b2, Wc, bc)
